# Initial kernel scaffold; baseline (speedup 1.0000x reference)
import jax
import jax.numpy as jnp
from jax.experimental import pallas as pl


def kernel(ids, linear_weights, embed_tables, dnn_proj, W1, b1, W2, b2, W3, b3, Wl, bl):
    raise NotImplementedError("write your pallas kernel here")


# speedup vs baseline: 45.3795x; 45.3795x over previous
import functools

import jax
import jax.numpy as jnp
from jax import lax
from jax.experimental import pallas as pl
from jax.experimental.pallas import tpu as pltpu
from jax.experimental.pallas import tpu_sc as plsc

B = 16384
F = 26
V = 100000
V2 = 100352
VB = V2 // 2
D = 32
H1, H2, H3 = 1024, 512, 256

NC, NS = 2, 16
NW = NC * NS
BPW = B // NW
CHUNK = 128
NJ = BPW // CHUNK
NCH = F * NJ
NG = CHUNK // 16
RING = 4


def _proj_body(w_ref, embt_ref, lin_ref, ptab_ref, ltab_ref):
    w = w_ref[0]
    x = embt_ref[0]
    ptab_ref[...] = jnp.sum(x * w.reshape(D, 1), axis=0)
    ltab_ref[...] = lin_ref[0, 0]


def _project(proj2, emb_t, lin2):
    return pl.pallas_call(
        _proj_body,
        grid=(F, V2 // VB),
        in_specs=[
            pl.BlockSpec((1, 1, D), lambda f, j: (f, 0, 0)),
            pl.BlockSpec((1, D, VB), lambda f, j: (f, 0, j)),
            pl.BlockSpec((1, 1, VB), lambda f, j: (f, 0, j)),
        ],
        out_specs=[
            pl.BlockSpec((VB,), lambda f, j: (f * (V2 // VB) + j,)),
            pl.BlockSpec((VB,), lambda f, j: (f * (V2 // VB) + j,)),
        ],
        out_shape=[
            jax.ShapeDtypeStruct((F * V2,), jnp.float32),
            jax.ShapeDtypeStruct((F * V2,), jnp.float32),
        ],
    )(proj2, emb_t, lin2)


def _sc_gather(ids_blk, ptab, ltab):
    mesh = plsc.VectorSubcoreMesh(core_axis_name="c", subcore_axis_name="s")

    @functools.partial(
        pl.kernel,
        out_type=(
            jax.ShapeDtypeStruct((B, F), jnp.float32),
            jax.ShapeDtypeStruct((B, F), jnp.float32),
        ),
        mesh=mesh,
        scratch_types=[
            pltpu.VMEM((F, NJ, CHUNK), jnp.int32),
            pltpu.VMEM((RING, CHUNK), jnp.float32),
            pltpu.VMEM((RING, CHUNK), jnp.float32),
            pltpu.VMEM((BPW, F), jnp.float32),
            pltpu.VMEM((BPW, F), jnp.float32),
            pltpu.SemaphoreType.DMA,
            pltpu.SemaphoreType.DMA,
        ],
        compiler_params=pltpu.CompilerParams(
            needs_layout_passes=False, use_tc_tiling_on_sc=False),
    )
    def sc_kernel(ids_hbm, ptab_hbm, ltab_hbm, dnn_out, lin_out,
                  ids_v, pbuf, lbuf, dnnb_v, linb_v, psem, lsem):
        wid = lax.axis_index("s") * NC + lax.axis_index("c")
        base = wid * BPW
        pltpu.sync_copy(ids_hbm.at[wid], ids_v)
        iot = lax.iota(jnp.int32, 16)

        def pcopy(c, s):
            f = c // NJ
            j = lax.rem(c, NJ)
            return pltpu.make_async_copy(
                ptab_hbm.at[ids_v.at[f, j]], pbuf.at[s], psem)

        def lcopy(c, s):
            f = c // NJ
            j = lax.rem(c, NJ)
            return pltpu.make_async_copy(
                ltab_hbm.at[ids_v.at[f, j]], lbuf.at[s], lsem)

        def fire(c, s):
            pcopy(c, s).start()
            lcopy(c, s).start()

        def wait(c, s):
            pcopy(c, s).wait()
            lcopy(c, s).wait()

        def consume(c, s):
            f = c // NJ
            j = lax.rem(c, NJ)
            fsp = jnp.full((16,), f, jnp.int32)
            rb = j * CHUNK
            for g in range(NG):
                ridx = iot + (rb + g * 16)
                plsc.store_scatter(dnnb_v, [ridx, fsp],
                                   pbuf.at[s][pl.ds(g * 16, 16)])
                plsc.store_scatter(linb_v, [ridx, fsp],
                                   lbuf.at[s][pl.ds(g * 16, 16)])

        for s in range(RING):
            fire(s, s)

        def body(q, carry):
            cb = RING * q
            for s in range(RING):
                c = cb + s
                wait(c, s)
                consume(c, s)
                fire(c + RING, s)
            return carry

        lax.fori_loop(0, NCH // RING - 1, body, 0)
        for s in range(RING):
            c = NCH - RING + s
            wait(c, s)
            consume(c, s)

        pltpu.sync_copy(dnnb_v, dnn_out.at[pl.ds(base, BPW)])
        pltpu.sync_copy(linb_v, lin_out.at[pl.ds(base, BPW)])

    return sc_kernel(ids_blk, ptab, ltab)


def _mlp_body(dnn_ref, lin_ref, w1_ref, b1_ref, w2_ref, b2_ref,
              w3_ref, b3_ref, wll_ref, wld_ref, bl_ref, out_ref):
    x = dnn_ref[...]
    h = jnp.maximum(
        jnp.dot(x, w1_ref[...], preferred_element_type=jnp.float32)
        + b1_ref[...], 0.0)
    h = jnp.maximum(
        jnp.dot(h, w2_ref[...], preferred_element_type=jnp.float32)
        + b2_ref[...], 0.0)
    h = jnp.maximum(
        jnp.dot(h, w3_ref[...], preferred_element_type=jnp.float32)
        + b3_ref[...], 0.0)
    out_ref[...] = (
        jnp.dot(h, wld_ref[...], preferred_element_type=jnp.float32)
        + jnp.dot(lin_ref[...], wll_ref[...],
                  preferred_element_type=jnp.float32)
        + bl_ref[...])


def _mlp(dnn_net, linear_net, W1, b1, W2, b2, W3, b3, Wl_lin, Wl_deep, bl,
         interpret=False):
    BM = 1024
    full = lambda shape: pl.BlockSpec(shape, lambda i: (0, 0))
    return pl.pallas_call(
        _mlp_body,
        grid=(B // BM,),
        in_specs=[
            pl.BlockSpec((BM, F), lambda i: (i, 0)),
            pl.BlockSpec((BM, F), lambda i: (i, 0)),
            full((F, H1)), full((1, H1)),
            full((H1, H2)), full((1, H2)),
            full((H2, H3)), full((1, H3)),
            full((F, 1)), full((H3, 1)), full((1, 1)),
        ],
        out_specs=pl.BlockSpec((BM, 1), lambda i: (i, 0)),
        out_shape=jax.ShapeDtypeStruct((B, 1), jnp.float32),
        interpret=interpret,
    )(dnn_net, linear_net, W1, b1.reshape(1, H1), W2, b2.reshape(1, H2),
      W3, b3.reshape(1, H3), Wl_lin, Wl_deep, bl.reshape(1, 1))


def kernel(ids, linear_weights, embed_tables, dnn_proj,
           W1, b1, W2, b2, W3, b3, Wl, bl):
    ids32 = ids.astype(jnp.int32)
    ids_off = ids32.T + (jnp.arange(F, dtype=jnp.int32) * V2)[:, None]
    ids_blk = (ids_off.reshape(F, NW, BPW)
               .transpose(1, 0, 2)
               .reshape(NW, F, NJ, CHUNK))
    emb_t = jnp.transpose(embed_tables, (0, 2, 1))
    lin2 = jnp.transpose(linear_weights, (0, 2, 1))
    proj2 = dnn_proj[..., 0].reshape(F, 1, D)

    ptab, ltab = _project(proj2, emb_t, lin2)
    dnn_net, linear_net = _sc_gather(ids_blk, ptab, ltab)
    return _mlp(dnn_net, linear_net, W1, b1, W2, b2, W3, b3,
                Wl[:F], Wl[F:], bl)
